# Initial kernel scaffold; baseline (speedup 1.0000x reference)
import jax
import jax.numpy as jnp
from jax.experimental import pallas as pl


def kernel(points, W1, b1, W2, b2, W3, b3, W4, b4, W5, b5, W6, b6):
    raise NotImplementedError("write your pallas kernel here")


# speedup vs baseline: 18.1497x; 18.1497x over previous
import jax
import jax.numpy as jnp
from jax.experimental import pallas as pl
from jax.experimental.pallas import tpu as pltpu

N = 4096
TILE = 256
NT = N // TILE
C1 = 64
C2 = 1024
H4 = 512
H5 = 256
NC = 6

_NT_DN = (((1,), (1,)), ((), ()))


def _stage_a(pt_ref, w1t_ref, b1_ref, w2_ref, b2_ref, w3_ref, b3_ref,
             w4a_ref, b4_ref, feat_ref, f2_ref, xx_ref, g_ref, gh_ref):
    i = pl.program_id(0)
    pt = pt_ref[...]
    f = (pt[:, 0:1] * w1t_ref[0:1, :] + pt[:, 1:2] * w1t_ref[1:2, :]
         + pt[:, 2:3] * w1t_ref[2:3, :]) + b1_ref[...]
    f = jnp.maximum(f, 0.0)
    f = jax.lax.dot_general(f, w2_ref[...], _NT_DN,
                            preferred_element_type=jnp.float32) + b2_ref[...]
    f = jnp.maximum(f, 0.0)
    f2 = jax.lax.dot_general(f, w3_ref[...], _NT_DN,
                             preferred_element_type=jnp.float32) + b3_ref[...]
    f2 = jnp.maximum(f2, 0.0)
    feat_ref[...] = f
    f2_ref[...] = f2
    xx_ref[...] = jnp.sum(f2 * f2, axis=1, keepdims=True)
    cur = jnp.max(f2, axis=0, keepdims=True)
    gnew = jnp.where(i == 0, cur, jnp.maximum(g_ref[...], cur))
    g_ref[...] = gnew

    @pl.when(i == NT - 1)
    def _():
        gh_ref[...] = jax.lax.dot_general(
            gnew, w4a_ref[...], _NT_DN,
            preferred_element_type=jnp.float32) + b4_ref[...]


def _stage_b(xi_ref, x_ref, xxt_ref, pt_t_ref, pt_ref, feat_ref, gh_ref,
             w4b_ref, w4ct_ref, w5_ref, b5_ref, w6_ref, b6_ref, out_ref):
    xi = xi_ref[...]
    inner = -2.0 * jax.lax.dot_general(xi, x_ref[...], _NT_DN,
                                       preferred_element_type=jnp.float32)
    xx_i = jnp.sum(xi * xi, axis=1, keepdims=True)
    d = (xx_i + inner) + xxt_ref[...]
    cols = jax.lax.broadcasted_iota(jnp.int32, (TILE, N), 1)
    acc = gh_ref[...] + jax.lax.dot_general(
        feat_ref[...], w4b_ref[...], _NT_DN,
        preferred_element_type=jnp.float32)
    neg_inf = jnp.float32(-jnp.inf)
    for r in range(4):
        m = jnp.max(d, axis=1, keepdims=True)
        j = jnp.min(jnp.where(d == m, cols, N), axis=1, keepdims=True)
        oh = cols == j
        if r > 0:
            nx = jnp.sum(jnp.where(oh, pt_t_ref[0:1, :], 0.0), axis=1,
                         keepdims=True)
            ny = jnp.sum(jnp.where(oh, pt_t_ref[1:2, :], 0.0), axis=1,
                         keepdims=True)
            nz = jnp.sum(jnp.where(oh, pt_t_ref[2:3, :], 0.0), axis=1,
                         keepdims=True)
            dx = nx - pt_ref[:, 0:1]
            dy = ny - pt_ref[:, 1:2]
            dz = nz - pt_ref[:, 2:3]
            base = 4 * (r - 1)
            acc = (acc + dx * w4ct_ref[base + 0:base + 1, :]
                   + dy * w4ct_ref[base + 1:base + 2, :]
                   + dz * w4ct_ref[base + 2:base + 3, :]
                   + (-m) * w4ct_ref[base + 3:base + 4, :])
        if r < 3:
            d = jnp.where(oh, neg_inf, d)
    h = jnp.maximum(acc, 0.0)
    h2 = jax.lax.dot_general(h, w5_ref[...], _NT_DN,
                             preferred_element_type=jnp.float32) + b5_ref[...]
    h2 = jnp.maximum(h2, 0.0)
    logits = jax.lax.dot_general(h2, w6_ref[...], _NT_DN,
                                 preferred_element_type=jnp.float32) + b6_ref[...]
    mx = jnp.max(logits, axis=1, keepdims=True)
    e = jnp.exp(logits - mx)
    out_ref[...] = e / jnp.sum(e, axis=1, keepdims=True)


def kernel(points, W1, b1, W2, b2, W3, b3, W4, b4, W5, b5, W6, b6):
    pts = points.reshape(N, 3)
    pts_t = pts.T
    w1t = W1.T
    w4a = W4[:, :C2]
    w4b = W4[:, C2:C2 + C1]
    w4ct = W4[:, C2 + C1:].T
    b1r = b1.reshape(1, -1)
    b2r = b2.reshape(1, -1)
    b3r = b3.reshape(1, -1)
    b4r = b4.reshape(1, -1)
    b5r = b5.reshape(1, -1)
    b6r = b6.reshape(1, -1)

    const = lambda i: (0, 0)
    row = lambda i: (i, 0)

    feat, f2, xx, _gmax, gh = pl.pallas_call(
        _stage_a,
        grid=(NT,),
        in_specs=[
            pl.BlockSpec((TILE, 3), row),
            pl.BlockSpec((3, C1), const),
            pl.BlockSpec((1, C1), const),
            pl.BlockSpec((C1, C1), const),
            pl.BlockSpec((1, C1), const),
            pl.BlockSpec((C2, C1), const),
            pl.BlockSpec((1, C2), const),
            pl.BlockSpec((H4, C2), const),
            pl.BlockSpec((1, H4), const),
        ],
        out_specs=[
            pl.BlockSpec((TILE, C1), row),
            pl.BlockSpec((TILE, C2), row),
            pl.BlockSpec((TILE, 1), row),
            pl.BlockSpec((1, C2), const),
            pl.BlockSpec((1, H4), const),
        ],
        out_shape=[
            jax.ShapeDtypeStruct((N, C1), jnp.float32),
            jax.ShapeDtypeStruct((N, C2), jnp.float32),
            jax.ShapeDtypeStruct((N, 1), jnp.float32),
            jax.ShapeDtypeStruct((1, C2), jnp.float32),
            jax.ShapeDtypeStruct((1, H4), jnp.float32),
        ],
        compiler_params=pltpu.CompilerParams(
            dimension_semantics=("arbitrary",)),
    )(pts, w1t, b1r, W2, b2r, W3, b3r, w4a, b4r)

    xxt = xx.reshape(1, N)

    preds = pl.pallas_call(
        _stage_b,
        grid=(NT,),
        in_specs=[
            pl.BlockSpec((TILE, C2), row),
            pl.BlockSpec((N, C2), const),
            pl.BlockSpec((1, N), const),
            pl.BlockSpec((3, N), const),
            pl.BlockSpec((TILE, 3), row),
            pl.BlockSpec((TILE, C1), row),
            pl.BlockSpec((1, H4), const),
            pl.BlockSpec((H4, C1), const),
            pl.BlockSpec((12, H4), const),
            pl.BlockSpec((H5, H4), const),
            pl.BlockSpec((1, H5), const),
            pl.BlockSpec((NC, H5), const),
            pl.BlockSpec((1, NC), const),
        ],
        out_specs=pl.BlockSpec((TILE, NC), row),
        out_shape=jax.ShapeDtypeStruct((N, NC), jnp.float32),
        compiler_params=pltpu.CompilerParams(
            dimension_semantics=("arbitrary",)),
    )(f2, f2, xxt, pts_t, pts, feat, gh, w4b, w4ct, W5, b5r, W6, b6r)

    return preds


# speedup vs baseline: 18.1696x; 1.0011x over previous
import jax
import jax.numpy as jnp
from jax.experimental import pallas as pl
from jax.experimental.pallas import tpu as pltpu

N = 4096
TILE = 256
NT = N // TILE
C1 = 64
C2 = 1024
H4 = 512
H5 = 256
NC = 6

_NT_DN = (((1,), (1,)), ((), ()))


def _stage_a(pt_ref, w1t_ref, b1_ref, w2_ref, b2_ref, w3_ref, b3_ref,
             w4a_ref, b4_ref, feat_ref, f2_ref, xx_ref, g_ref, gh_ref):
    i = pl.program_id(0)
    pt = pt_ref[...]
    f = (pt[:, 0:1] * w1t_ref[0:1, :] + pt[:, 1:2] * w1t_ref[1:2, :]
         + pt[:, 2:3] * w1t_ref[2:3, :]) + b1_ref[...]
    f = jnp.maximum(f, 0.0)
    f = jax.lax.dot_general(f, w2_ref[...], _NT_DN,
                            preferred_element_type=jnp.float32) + b2_ref[...]
    f = jnp.maximum(f, 0.0)
    f2 = jax.lax.dot_general(f, w3_ref[...], _NT_DN,
                             preferred_element_type=jnp.float32) + b3_ref[...]
    f2 = jnp.maximum(f2, 0.0)
    feat_ref[...] = f
    f2_ref[...] = f2
    xx_ref[...] = jnp.sum(f2 * f2, axis=1, keepdims=True)
    cur = jnp.max(f2, axis=0, keepdims=True)
    gnew = jnp.where(i == 0, cur, jnp.maximum(g_ref[...], cur))
    g_ref[...] = gnew

    @pl.when(i == NT - 1)
    def _():
        gh_ref[...] = jax.lax.dot_general(
            gnew, w4a_ref[...], _NT_DN,
            preferred_element_type=jnp.float32) + b4_ref[...]


def _stage_b(xi_ref, x_ref, xxt_ref, pt_t_ref, pt_ref, feat_ref, gh_ref,
             w4b_ref, w4ct_ref, w5_ref, b5_ref, w6_ref, b6_ref, out_ref):
    xi = xi_ref[...]
    inner = -2.0 * jax.lax.dot_general(xi, x_ref[...], _NT_DN,
                                       preferred_element_type=jnp.float32)
    xx_i = jnp.sum(xi * xi, axis=1, keepdims=True)
    d = (xx_i + inner) + xxt_ref[...]
    cols = jax.lax.broadcasted_iota(jnp.int32, (TILE, N), 1)
    acc = gh_ref[...] + jax.lax.dot_general(
        feat_ref[...], w4b_ref[...], _NT_DN,
        preferred_element_type=jnp.float32)
    neg_inf = jnp.float32(-jnp.inf)
    for r in range(4):
        m = jnp.max(d, axis=1, keepdims=True)
        j = jnp.min(jnp.where(d == m, cols, N), axis=1, keepdims=True)
        oh = cols == j
        if r > 0:
            nx = jnp.sum(jnp.where(oh, pt_t_ref[0:1, :], 0.0), axis=1,
                         keepdims=True)
            ny = jnp.sum(jnp.where(oh, pt_t_ref[1:2, :], 0.0), axis=1,
                         keepdims=True)
            nz = jnp.sum(jnp.where(oh, pt_t_ref[2:3, :], 0.0), axis=1,
                         keepdims=True)
            dx = nx - pt_ref[:, 0:1]
            dy = ny - pt_ref[:, 1:2]
            dz = nz - pt_ref[:, 2:3]
            base = 4 * (r - 1)
            acc = (acc + dx * w4ct_ref[base + 0:base + 1, :]
                   + dy * w4ct_ref[base + 1:base + 2, :]
                   + dz * w4ct_ref[base + 2:base + 3, :]
                   + (-m) * w4ct_ref[base + 3:base + 4, :])
        if r < 3:
            d = jnp.where(oh, neg_inf, d)
    h = jnp.maximum(acc, 0.0)
    h2 = jax.lax.dot_general(h, w5_ref[...], _NT_DN,
                             preferred_element_type=jnp.float32) + b5_ref[...]
    h2 = jnp.maximum(h2, 0.0)
    logits = jax.lax.dot_general(h2, w6_ref[...], _NT_DN,
                                 preferred_element_type=jnp.float32) + b6_ref[...]
    mx = jnp.max(logits, axis=1, keepdims=True)
    e = jnp.exp(logits - mx)
    out_ref[...] = e / jnp.sum(e, axis=1, keepdims=True)


def kernel(points, W1, b1, W2, b2, W3, b3, W4, b4, W5, b5, W6, b6):
    pts = points.reshape(N, 3)
    pts_t = pts.T
    w1t = W1.T
    w4a = W4[:, :C2]
    w4b = W4[:, C2:C2 + C1]
    w4ct = W4[:, C2 + C1:].T
    b1r = b1.reshape(1, -1)
    b2r = b2.reshape(1, -1)
    b3r = b3.reshape(1, -1)
    b4r = b4.reshape(1, -1)
    b5r = b5.reshape(1, -1)
    b6r = b6.reshape(1, -1)

    const = lambda i: (0, 0)
    row = lambda i: (i, 0)

    feat, f2, xx, _gmax, gh = pl.pallas_call(
        _stage_a,
        grid=(NT,),
        in_specs=[
            pl.BlockSpec((TILE, 3), row),
            pl.BlockSpec((3, C1), const),
            pl.BlockSpec((1, C1), const),
            pl.BlockSpec((C1, C1), const),
            pl.BlockSpec((1, C1), const),
            pl.BlockSpec((C2, C1), const),
            pl.BlockSpec((1, C2), const),
            pl.BlockSpec((H4, C2), const),
            pl.BlockSpec((1, H4), const),
        ],
        out_specs=[
            pl.BlockSpec((TILE, C1), row),
            pl.BlockSpec((TILE, C2), row),
            pl.BlockSpec((TILE, 1), row),
            pl.BlockSpec((1, C2), const),
            pl.BlockSpec((1, H4), const),
        ],
        out_shape=[
            jax.ShapeDtypeStruct((N, C1), jnp.float32),
            jax.ShapeDtypeStruct((N, C2), jnp.float32),
            jax.ShapeDtypeStruct((N, 1), jnp.float32),
            jax.ShapeDtypeStruct((1, C2), jnp.float32),
            jax.ShapeDtypeStruct((1, H4), jnp.float32),
        ],
        compiler_params=pltpu.CompilerParams(
            dimension_semantics=("arbitrary",)),
    )(pts, w1t, b1r, W2, b2r, W3, b3r, w4a, b4r)

    xxt = xx.reshape(1, N)

    preds = pl.pallas_call(
        _stage_b,
        grid=(NT,),
        in_specs=[
            pl.BlockSpec((TILE, C2), row),
            pl.BlockSpec((N, C2), const),
            pl.BlockSpec((1, N), const),
            pl.BlockSpec((3, N), const),
            pl.BlockSpec((TILE, 3), row),
            pl.BlockSpec((TILE, C1), row),
            pl.BlockSpec((1, H4), const),
            pl.BlockSpec((H4, C1), const),
            pl.BlockSpec((12, H4), const),
            pl.BlockSpec((H5, H4), const),
            pl.BlockSpec((1, H5), const),
            pl.BlockSpec((NC, H5), const),
            pl.BlockSpec((1, NC), const),
        ],
        out_specs=pl.BlockSpec((TILE, NC), row),
        out_shape=jax.ShapeDtypeStruct((N, NC), jnp.float32),
        compiler_params=pltpu.CompilerParams(
            dimension_semantics=("parallel",)),
    )(f2, f2, xxt, pts_t, pts, feat, gh, w4b, w4ct, W5, b5r, W6, b6r)

    return preds


# speedup vs baseline: 21.2826x; 1.1713x over previous
import jax
import jax.numpy as jnp
from jax.experimental import pallas as pl
from jax.experimental.pallas import tpu as pltpu

N = 4096
TILE = 256
NT = N // TILE
C1 = 64
C2 = 1024
H4 = 512
H5 = 256
NC = 6

_NT_DN = (((1,), (1,)), ((), ()))


def _stage_a(pt_ref, w1t_ref, b1_ref, w2_ref, b2_ref, w3_ref, b3_ref,
             feat_ref, f2_ref, xx_ref, tmax_ref):
    pt = pt_ref[...]
    f = (pt[:, 0:1] * w1t_ref[0:1, :] + pt[:, 1:2] * w1t_ref[1:2, :]
         + pt[:, 2:3] * w1t_ref[2:3, :]) + b1_ref[...]
    f = jnp.maximum(f, 0.0)
    f = jax.lax.dot_general(f, w2_ref[...], _NT_DN,
                            preferred_element_type=jnp.float32) + b2_ref[...]
    f = jnp.maximum(f, 0.0)
    f2 = jax.lax.dot_general(f, w3_ref[...], _NT_DN,
                             preferred_element_type=jnp.float32) + b3_ref[...]
    f2 = jnp.maximum(f2, 0.0)
    feat_ref[...] = f
    f2_ref[...] = f2
    xx_ref[...] = jnp.sum(f2 * f2, axis=1, keepdims=True)
    tmax_ref[...] = jnp.max(f2, axis=0, keepdims=True).reshape(1, 1, C2)


_NN_DN = (((1,), (0,)), ((), ()))


def _stage_b(xi_ref, x_ref, xxt_ref, p_ref, pt_ref, feat_ref, tmax_ref,
             w4a_ref, b4_ref, w4b_ref, w4ct_ref, w5_ref, b5_ref, w6_ref,
             b6_ref, out_ref):
    xi = xi_ref[...]
    inner = -2.0 * jax.lax.dot_general(xi, x_ref[...], _NT_DN,
                                       preferred_element_type=jnp.float32)
    xx_i = jnp.sum(xi * xi, axis=1, keepdims=True)
    d = (xx_i + inner) + xxt_ref[...]
    cols = jax.lax.broadcasted_iota(jnp.int32, (TILE, N), 1)
    gmax = jnp.max(tmax_ref[...], axis=0)
    gh = jax.lax.dot_general(gmax, w4a_ref[...], _NT_DN,
                             preferred_element_type=jnp.float32) + b4_ref[...]
    acc = gh + jax.lax.dot_general(
        feat_ref[...], w4b_ref[...], _NT_DN,
        preferred_element_type=jnp.float32)
    neg_inf = jnp.float32(-jnp.inf)
    for r in range(4):
        m = jnp.max(d, axis=1, keepdims=True)
        j = jnp.min(jnp.where(d == m, cols, N), axis=1, keepdims=True)
        oh = cols == j
        if r > 0:
            nbr = jax.lax.dot_general(oh.astype(jnp.float32), p_ref[...],
                                      _NN_DN,
                                      preferred_element_type=jnp.float32)
            dx = nbr[:, 0:1] - pt_ref[:, 0:1]
            dy = nbr[:, 1:2] - pt_ref[:, 1:2]
            dz = nbr[:, 2:3] - pt_ref[:, 2:3]
            base = 4 * (r - 1)
            acc = (acc + dx * w4ct_ref[base + 0:base + 1, :]
                   + dy * w4ct_ref[base + 1:base + 2, :]
                   + dz * w4ct_ref[base + 2:base + 3, :]
                   + (-m) * w4ct_ref[base + 3:base + 4, :])
        if r < 3:
            d = jnp.where(oh, neg_inf, d)
    h = jnp.maximum(acc, 0.0)
    h2 = jax.lax.dot_general(h, w5_ref[...], _NT_DN,
                             preferred_element_type=jnp.float32) + b5_ref[...]
    h2 = jnp.maximum(h2, 0.0)
    logits = jax.lax.dot_general(h2, w6_ref[...], _NT_DN,
                                 preferred_element_type=jnp.float32) + b6_ref[...]
    mx = jnp.max(logits, axis=1, keepdims=True)
    e = jnp.exp(logits - mx)
    out_ref[...] = e / jnp.sum(e, axis=1, keepdims=True)


def kernel(points, W1, b1, W2, b2, W3, b3, W4, b4, W5, b5, W6, b6):
    pts = points.reshape(N, 3)
    w1t = W1.T
    w4a = W4[:, :C2]
    w4b = W4[:, C2:C2 + C1]
    w4ct = W4[:, C2 + C1:].T
    b1r = b1.reshape(1, -1)
    b2r = b2.reshape(1, -1)
    b3r = b3.reshape(1, -1)
    b4r = b4.reshape(1, -1)
    b5r = b5.reshape(1, -1)
    b6r = b6.reshape(1, -1)

    const = lambda i: (0, 0)
    row = lambda i: (i, 0)

    feat, f2, xx, tmax = pl.pallas_call(
        _stage_a,
        grid=(NT,),
        in_specs=[
            pl.BlockSpec((TILE, 3), row),
            pl.BlockSpec((3, C1), const),
            pl.BlockSpec((1, C1), const),
            pl.BlockSpec((C1, C1), const),
            pl.BlockSpec((1, C1), const),
            pl.BlockSpec((C2, C1), const),
            pl.BlockSpec((1, C2), const),
        ],
        out_specs=[
            pl.BlockSpec((TILE, C1), row),
            pl.BlockSpec((TILE, C2), row),
            pl.BlockSpec((TILE, 1), row),
            pl.BlockSpec((1, 1, C2), lambda i: (i, 0, 0)),
        ],
        out_shape=[
            jax.ShapeDtypeStruct((N, C1), jnp.float32),
            jax.ShapeDtypeStruct((N, C2), jnp.float32),
            jax.ShapeDtypeStruct((N, 1), jnp.float32),
            jax.ShapeDtypeStruct((NT, 1, C2), jnp.float32),
        ],
        compiler_params=pltpu.CompilerParams(
            dimension_semantics=("parallel",)),
    )(pts, w1t, b1r, W2, b2r, W3, b3r)

    xxt = xx.reshape(1, N)

    preds = pl.pallas_call(
        _stage_b,
        grid=(NT,),
        in_specs=[
            pl.BlockSpec((TILE, C2), row),
            pl.BlockSpec((N, C2), const),
            pl.BlockSpec((1, N), const),
            pl.BlockSpec((N, 3), const),
            pl.BlockSpec((TILE, 3), row),
            pl.BlockSpec((TILE, C1), row),
            pl.BlockSpec((NT, 1, C2), lambda i: (0, 0, 0)),
            pl.BlockSpec((H4, C2), const),
            pl.BlockSpec((1, H4), const),
            pl.BlockSpec((H4, C1), const),
            pl.BlockSpec((12, H4), const),
            pl.BlockSpec((H5, H4), const),
            pl.BlockSpec((1, H5), const),
            pl.BlockSpec((NC, H5), const),
            pl.BlockSpec((1, NC), const),
        ],
        out_specs=pl.BlockSpec((TILE, NC), row),
        out_shape=jax.ShapeDtypeStruct((N, NC), jnp.float32),
        compiler_params=pltpu.CompilerParams(
            dimension_semantics=("parallel",)),
    )(f2, f2, xxt, pts, pts, feat, tmax, w4a, b4r, w4b, w4ct, W5, b5r,
      W6, b6r)

    return preds
